# baseline (speedup 1.0000x reference)
import functools

import jax
import jax.numpy as jnp
from jax import lax
from jax.experimental import pallas as pl
from jax.experimental.pallas import tpu as pltpu
from jax.experimental.pallas import tpu_sc as plsc

N = 10000
E = 320000
M = 1000
D = 128

_NC = 2
_NS = 16
_NW = _NC * _NS
_CHUNK = 64
_NBUF = 5
_QCHUNKS = 20
_QEDGES = _QCHUNKS * _CHUNK
_NSEC = E // _QEDGES
_SEC_PER_W = -(-_NSEC // _NW)
_N_PAD = 10240
_ROWS_PER_SUB = _N_PAD // _NS

_BR = 400
_NB = N // _BR
_BR2 = 2000
_NB2 = N // _BR2


def _sc_segment_sum(feat_a, edge_index):
    mesh = plsc.VectorSubcoreMesh(core_axis_name="c", subcore_axis_name="s")

    @functools.partial(
        pl.kernel,
        out_type=jax.ShapeDtypeStruct((_NC, _N_PAD, D), jnp.float32),
        mesh=mesh,
        scratch_types=[
            pltpu.VMEM_SHARED((_N_PAD, D), jnp.float32),
        ]
        + [pltpu.VMEM((_QEDGES,), jnp.int32) for _ in range(4)]
        + [pltpu.VMEM((_CHUNK, D), jnp.float32) for _ in range(_NBUF)]
        + [pltpu.VMEM((_CHUNK,), jnp.int32) for _ in range(2 * _NBUF)]
        + [pltpu.SemaphoreType.DMA for _ in range(2 * _NBUF + 2)],
    )
    def seg_sum(feat_hbm, edge_hbm, out_hbm, acc_sh, *bufs):
        srcq = bufs[0:2]
        dstq = bufs[2:4]
        rows = bufs[4:4 + _NBUF]
        srcb = bufs[4 + _NBUF:4 + 2 * _NBUF]
        dstb = bufs[4 + 2 * _NBUF:4 + 3 * _NBUF]
        sems = bufs[4 + 3 * _NBUF:]
        gsem = sems[:_NBUF]
        ssem = sems[_NBUF:2 * _NBUF]
        isem = sems[2 * _NBUF:]
        c = lax.axis_index("c")
        s = lax.axis_index("s")
        w = s * _NC + c

        @pl.loop(0, _CHUNK)
        def _(r):
            @pl.loop(0, D // 16)
            def _(k):
                rows[0][r, pl.ds(k * 16, 16)] = jnp.zeros((16,), jnp.float32)

        @pl.loop(0, _ROWS_PER_SUB // _CHUNK)
        def _(k):
            pltpu.sync_copy(
                rows[0], acc_sh.at[pl.ds(s * _ROWS_PER_SUB + k * _CHUNK, _CHUNK)])

        def load_idx(u, qs, sync):
            off = u * _QEDGES
            for row, vm in ((0, srcq[qs]), (1, dstq[qs])):
                cp = pltpu.make_async_copy(
                    edge_hbm.at[row].at[pl.ds(off, _QEDGES)], vm, isem[qs])
                cp.start()
                if sync:
                    cp.wait()

        def wait_idx(u, qs):
            off = u * _QEDGES
            for row, vm in ((0, srcq[qs]), (1, dstq[qs])):
                pltpu.make_async_copy(
                    edge_hbm.at[row].at[pl.ds(off, _QEDGES)], vm,
                    isem[qs]).wait()

        load_idx(w, 0, True)

        plsc.subcore_barrier()

        def stage_idx(slab, buf, j):
            for k in range(_CHUNK // 16):
                buf[pl.ds(k * 16, 16)] = slab[pl.ds(j * _CHUNK + k * 16, 16)]

        def start_gather(sq, j, b):
            stage_idx(sq, srcb[b], j)
            pltpu.async_copy(feat_hbm.at[srcb[b]], rows[b], gsem[b])

        def wait_gather(b):
            pltpu.make_async_copy(feat_hbm.at[srcb[b]], rows[b],
                                  gsem[b]).wait()

        def start_scatter(dq, j, b):
            stage_idx(dq, dstb[b], j)
            pltpu.async_copy(rows[b], acc_sh.at[dstb[b]], ssem[b], add=True)

        def wait_scatter(b):
            pltpu.make_async_copy(rows[b], acc_sh.at[dstb[b]], ssem[b]).wait()

        for t in range(_SEC_PER_W):
            qs = t % 2
            sq, dq = srcq[qs], dstq[qs]
            u = w + _NW * t

            @pl.when(u < _NSEC)
            def _():
                if t > 0:
                    wait_idx(u, qs)
                if t + 1 < _SEC_PER_W:
                    nxt = w + _NW * (t + 1)

                    @pl.when(nxt < _NSEC)
                    def _():
                        load_idx(nxt, 1 - qs, False)

                for b in range(_NBUF - 1):
                    start_gather(sq, b, b)

                @pl.loop(0, _QCHUNKS // _NBUF)
                def _(g):
                    for b in range(_NBUF):
                        j = g * _NBUF + b
                        prev_slot = (b - 1) % _NBUF
                        wait_gather(b)

                        @pl.when(j >= 1)
                        def _():
                            wait_scatter(prev_slot)

                        @pl.when(j + _NBUF - 1 < _QCHUNKS)
                        def _():
                            start_gather(sq, j + _NBUF - 1, prev_slot)

                        start_scatter(dq, j, b)

                wait_scatter((_QCHUNKS - 1) % _NBUF)

        plsc.subcore_barrier()

        pltpu.sync_copy(
            acc_sh.at[pl.ds(s * _ROWS_PER_SUB, _ROWS_PER_SUB)],
            out_hbm.at[c].at[pl.ds(s * _ROWS_PER_SUB, _ROWS_PER_SUB)],
        )

    return seg_sum(feat_a, edge_index)


def _tc_partial(feat_a, deg, pm_pd, feat_b, W_prev, W_deg, W_fuse):
    def body(fa, dg, pp, fb, wp, wd, wf, out, fb_proj):
        i = pl.program_id(0)

        @pl.when(i == 0)
        def _():
            fb_proj[...] = jnp.dot(fb[...], wf[...],
                                   preferred_element_type=jnp.float32)

        acc = jnp.dot(fa[...], wp[...], preferred_element_type=jnp.float32)
        acc += jnp.dot(dg[...] * fa[...], wd[...],
                       preferred_element_type=jnp.float32)
        acc += jnp.dot(pp[...], fb_proj[...], preferred_element_type=jnp.float32)
        out[...] = acc

    return pl.pallas_call(
        body,
        grid=(_NB,),
        in_specs=[
            pl.BlockSpec((_BR, D), lambda i: (i, 0)),
            pl.BlockSpec((_BR, 1), lambda i: (i, 0)),
            pl.BlockSpec((_BR, M), lambda i: (i, 0)),
            pl.BlockSpec((M, D), lambda i: (0, 0)),
            pl.BlockSpec((D, D), lambda i: (0, 0)),
            pl.BlockSpec((D, D), lambda i: (0, 0)),
            pl.BlockSpec((D, D), lambda i: (0, 0)),
        ],
        out_specs=pl.BlockSpec((_BR, D), lambda i: (i, 0)),
        out_shape=jax.ShapeDtypeStruct((N, D), jnp.float32),
        scratch_shapes=[pltpu.VMEM((M, D), jnp.float32)],
    )(feat_a, deg, pm_pd, feat_b, W_prev, W_deg, W_fuse)


def _tc_combine(partial, agg2, W_gcn, bias):
    def body(pt, a01, wg, b, r_out, st_out, acc):
        i = pl.program_id(0)
        a = a01[0] + a01[1]
        r = pt[...] + jnp.dot(a, wg[...], preferred_element_type=jnp.float32)
        r = r + b[...]
        r_out[...] = r

        @pl.when(i == 0)
        def _():
            acc[...] = jnp.zeros_like(acc)

        acc[0:1, :] += jnp.sum(r, axis=0, keepdims=True)
        acc[1:2, :] += jnp.sum(r * r, axis=0, keepdims=True)

        @pl.when(i == _NB2 - 1)
        def _():
            st_out[...] = acc[...]

    return pl.pallas_call(
        body,
        grid=(_NB2,),
        in_specs=[
            pl.BlockSpec((_BR2, D), lambda i: (i, 0)),
            pl.BlockSpec((_NC, _BR2, D), lambda i: (0, i, 0)),
            pl.BlockSpec((D, D), lambda i: (0, 0)),
            pl.BlockSpec((1, D), lambda i: (0, 0)),
        ],
        out_specs=[
            pl.BlockSpec((_BR2, D), lambda i: (i, 0)),
            pl.BlockSpec((8, D), lambda i: (0, 0)),
        ],
        out_shape=[
            jax.ShapeDtypeStruct((N, D), jnp.float32),
            jax.ShapeDtypeStruct((8, D), jnp.float32),
        ],
        scratch_shapes=[pltpu.VMEM((8, D), jnp.float32)],
    )(partial, agg2, W_gcn, bias)


def _tc_norm(r, stats, gamma, beta):
    def body(r_ref, st, g, bt, out, coef):
        i = pl.program_id(0)

        @pl.when(i == 0)
        def _():
            mean = st[0:1, :] * (1.0 / N)
            var = st[1:2, :] * (1.0 / N) - mean * mean
            scale = g[...] * lax.rsqrt(var + 1e-5)
            coef[0:1, :] = scale
            coef[1:2, :] = bt[...] - mean * scale

        out[...] = r_ref[...] * coef[0:1, :] + coef[1:2, :]

    return pl.pallas_call(
        body,
        grid=(_NB2,),
        in_specs=[
            pl.BlockSpec((_BR2, D), lambda i: (i, 0)),
            pl.BlockSpec((8, D), lambda i: (0, 0)),
            pl.BlockSpec((1, D), lambda i: (0, 0)),
            pl.BlockSpec((1, D), lambda i: (0, 0)),
        ],
        out_specs=pl.BlockSpec((_BR2, D), lambda i: (i, 0)),
        out_shape=jax.ShapeDtypeStruct((N, D), jnp.float32),
        scratch_shapes=[pltpu.VMEM((8, D), jnp.float32)],
    )(r, stats, gamma, beta)


def kernel(feat_a, feat_b, deg, pm_pd, edge_index, W_prev, b_prev, W_deg,
           b_deg, W_gcn, b_gcn, W_fuse, b_fuse, bn_gamma, bn_beta):
    agg2 = _sc_segment_sum(feat_a, edge_index)
    partial = _tc_partial(feat_a, deg, pm_pd, feat_b, W_prev, W_deg, W_fuse)
    bias = (b_prev + b_deg + b_gcn + b_fuse).reshape(1, D)
    r, stats = _tc_combine(partial, agg2, W_gcn, bias)
    return _tc_norm(r, stats, bn_gamma.reshape(1, D), bn_beta.reshape(1, D))


# speedup vs baseline: 1.0640x; 1.0640x over previous
import functools

import jax
import jax.numpy as jnp
from jax import lax
from jax.experimental import pallas as pl
from jax.experimental.pallas import tpu as pltpu
from jax.experimental.pallas import tpu_sc as plsc

N = 10000
E = 320000
M = 1000
D = 128

_NC = 2
_NS = 16
_NW = _NC * _NS
_CHUNK = 64
_NBUF = 4
_QCHUNKS = 40
_QEDGES = _QCHUNKS * _CHUNK
_NSEC = E // _QEDGES
_SEC_PER_W = -(-_NSEC // _NW)
_N_PAD = 10240
_ROWS_PER_SUB = _N_PAD // _NS

_BR = 400
_NB = N // _BR
_BR2 = 2000
_NB2 = N // _BR2


def _sc_segment_sum(feat_a, edge_index):
    mesh = plsc.VectorSubcoreMesh(core_axis_name="c", subcore_axis_name="s")

    @functools.partial(
        pl.kernel,
        out_type=jax.ShapeDtypeStruct((_NC, _N_PAD, D), jnp.float32),
        mesh=mesh,
        scratch_types=[
            pltpu.VMEM_SHARED((_N_PAD, D), jnp.float32),
        ]
        + [pltpu.VMEM((_QEDGES,), jnp.int32) for _ in range(4)]
        + [pltpu.VMEM((_CHUNK, D), jnp.float32) for _ in range(_NBUF)]
        + [pltpu.VMEM((_CHUNK,), jnp.int32) for _ in range(2 * _NBUF)]
        + [pltpu.SemaphoreType.DMA for _ in range(2 * _NBUF + 2)],
    )
    def seg_sum(feat_hbm, edge_hbm, out_hbm, acc_sh, *bufs):
        srcq = bufs[0:2]
        dstq = bufs[2:4]
        rows = bufs[4:4 + _NBUF]
        srcb = bufs[4 + _NBUF:4 + 2 * _NBUF]
        dstb = bufs[4 + 2 * _NBUF:4 + 3 * _NBUF]
        sems = bufs[4 + 3 * _NBUF:]
        gsem = sems[:_NBUF]
        ssem = sems[_NBUF:2 * _NBUF]
        isem = sems[2 * _NBUF:]
        c = lax.axis_index("c")
        s = lax.axis_index("s")
        w = s * _NC + c

        @pl.loop(0, _CHUNK)
        def _(r):
            @pl.loop(0, D // 16)
            def _(k):
                rows[0][r, pl.ds(k * 16, 16)] = jnp.zeros((16,), jnp.float32)

        @pl.loop(0, _ROWS_PER_SUB // _CHUNK)
        def _(k):
            pltpu.sync_copy(
                rows[0], acc_sh.at[pl.ds(s * _ROWS_PER_SUB + k * _CHUNK, _CHUNK)])

        def load_idx(u, qs, sync):
            off = u * _QEDGES
            for row, vm in ((0, srcq[qs]), (1, dstq[qs])):
                cp = pltpu.make_async_copy(
                    edge_hbm.at[row].at[pl.ds(off, _QEDGES)], vm, isem[qs])
                cp.start()
                if sync:
                    cp.wait()

        def wait_idx(u, qs):
            off = u * _QEDGES
            for row, vm in ((0, srcq[qs]), (1, dstq[qs])):
                pltpu.make_async_copy(
                    edge_hbm.at[row].at[pl.ds(off, _QEDGES)], vm,
                    isem[qs]).wait()

        load_idx(w, 0, True)

        plsc.subcore_barrier()

        def stage_idx(slab, buf, j):
            for k in range(_CHUNK // 16):
                buf[pl.ds(k * 16, 16)] = slab[pl.ds(j * _CHUNK + k * 16, 16)]

        def start_gather(sq, j, b):
            stage_idx(sq, srcb[b], j)
            pltpu.async_copy(feat_hbm.at[srcb[b]], rows[b], gsem[b])

        def wait_gather(b):
            pltpu.make_async_copy(feat_hbm.at[srcb[b]], rows[b],
                                  gsem[b]).wait()

        def start_scatter(dq, j, b):
            stage_idx(dq, dstb[b], j)
            pltpu.async_copy(rows[b], acc_sh.at[dstb[b]], ssem[b], add=True)

        def wait_scatter(b):
            pltpu.make_async_copy(rows[b], acc_sh.at[dstb[b]], ssem[b]).wait()

        for t in range(_SEC_PER_W):
            qs = t % 2
            sq, dq = srcq[qs], dstq[qs]
            u = w + _NW * t

            @pl.when(u < _NSEC)
            def _():
                if t > 0:
                    wait_idx(u, qs)
                if t + 1 < _SEC_PER_W:
                    nxt = w + _NW * (t + 1)

                    @pl.when(nxt < _NSEC)
                    def _():
                        load_idx(nxt, 1 - qs, False)

                for b in range(_NBUF - 1):
                    start_gather(sq, b, b)

                @pl.loop(0, _QCHUNKS // _NBUF)
                def _(g):
                    for b in range(_NBUF):
                        j = g * _NBUF + b
                        prev_slot = (b - 1) % _NBUF
                        wait_gather(b)

                        @pl.when(j >= 1)
                        def _():
                            wait_scatter(prev_slot)

                        @pl.when(j + _NBUF - 1 < _QCHUNKS)
                        def _():
                            start_gather(sq, j + _NBUF - 1, prev_slot)

                        start_scatter(dq, j, b)

                wait_scatter((_QCHUNKS - 1) % _NBUF)

        plsc.subcore_barrier()

        pltpu.sync_copy(
            acc_sh.at[pl.ds(s * _ROWS_PER_SUB, _ROWS_PER_SUB)],
            out_hbm.at[c].at[pl.ds(s * _ROWS_PER_SUB, _ROWS_PER_SUB)],
        )

    return seg_sum(feat_a, edge_index)


def _tc_partial(feat_a, deg, pm_pd, feat_b, W_prev, W_deg, W_fuse):
    def body(fa, dg, pp, fb, wp, wd, wf, out, fb_proj):
        i = pl.program_id(0)

        @pl.when(i == 0)
        def _():
            fb_proj[...] = jnp.dot(fb[...], wf[...],
                                   preferred_element_type=jnp.float32)

        acc = jnp.dot(fa[...], wp[...], preferred_element_type=jnp.float32)
        acc += jnp.dot(dg[...] * fa[...], wd[...],
                       preferred_element_type=jnp.float32)
        acc += jnp.dot(pp[...], fb_proj[...], preferred_element_type=jnp.float32)
        out[...] = acc

    return pl.pallas_call(
        body,
        grid=(_NB,),
        in_specs=[
            pl.BlockSpec((_BR, D), lambda i: (i, 0)),
            pl.BlockSpec((_BR, 1), lambda i: (i, 0)),
            pl.BlockSpec((_BR, M), lambda i: (i, 0)),
            pl.BlockSpec((M, D), lambda i: (0, 0)),
            pl.BlockSpec((D, D), lambda i: (0, 0)),
            pl.BlockSpec((D, D), lambda i: (0, 0)),
            pl.BlockSpec((D, D), lambda i: (0, 0)),
        ],
        out_specs=pl.BlockSpec((_BR, D), lambda i: (i, 0)),
        out_shape=jax.ShapeDtypeStruct((N, D), jnp.float32),
        scratch_shapes=[pltpu.VMEM((M, D), jnp.float32)],
    )(feat_a, deg, pm_pd, feat_b, W_prev, W_deg, W_fuse)


def _tc_finish(partial, agg2, W_gcn, bias, gamma, beta):
    def body(pt, a01, wg, b, g, bt, out, acc):
        i = pl.program_id(0)
        a = a01[0] + a01[1]
        r = pt[...] + jnp.dot(a, wg[...], preferred_element_type=jnp.float32)
        r = r + b[...]
        out[pl.ds(i * _BR2, _BR2), :] = r

        @pl.when(i == 0)
        def _():
            acc[...] = jnp.zeros_like(acc)

        acc[0:1, :] += jnp.sum(r, axis=0, keepdims=True)
        acc[1:2, :] += jnp.sum(r * r, axis=0, keepdims=True)

        @pl.when(i == _NB2 - 1)
        def _():
            mean = acc[0:1, :] * (1.0 / N)
            var = acc[1:2, :] * (1.0 / N) - mean * mean
            scale = g[...] * lax.rsqrt(var + 1e-5)
            shift = bt[...] - mean * scale
            out[...] = out[...] * scale + shift

    return pl.pallas_call(
        body,
        grid=(_NB2,),
        in_specs=[
            pl.BlockSpec((_BR2, D), lambda i: (i, 0)),
            pl.BlockSpec((_NC, _BR2, D), lambda i: (0, i, 0)),
            pl.BlockSpec((D, D), lambda i: (0, 0)),
            pl.BlockSpec((1, D), lambda i: (0, 0)),
            pl.BlockSpec((1, D), lambda i: (0, 0)),
            pl.BlockSpec((1, D), lambda i: (0, 0)),
        ],
        out_specs=pl.BlockSpec((N, D), lambda i: (0, 0)),
        out_shape=jax.ShapeDtypeStruct((N, D), jnp.float32),
        scratch_shapes=[pltpu.VMEM((8, D), jnp.float32)],
    )(partial, agg2, W_gcn, bias, gamma, beta)


def kernel(feat_a, feat_b, deg, pm_pd, edge_index, W_prev, b_prev, W_deg,
           b_deg, W_gcn, b_gcn, W_fuse, b_fuse, bn_gamma, bn_beta):
    agg2 = _sc_segment_sum(feat_a, edge_index)
    partial = _tc_partial(feat_a, deg, pm_pd, feat_b, W_prev, W_deg, W_fuse)
    bias = (b_prev + b_deg + b_gcn + b_fuse).reshape(1, D)
    return _tc_finish(partial, agg2, W_gcn, bias,
                      bn_gamma.reshape(1, D), bn_beta.reshape(1, D))


# speedup vs baseline: 1.0737x; 1.0091x over previous
import functools

import jax
import jax.numpy as jnp
from jax import lax
from jax.experimental import pallas as pl
from jax.experimental.pallas import tpu as pltpu
from jax.experimental.pallas import tpu_sc as plsc

N = 10000
E = 320000
M = 1000
D = 128

_NC = 2
_NS = 16
_NW = _NC * _NS
_CHUNK = 64
_NBUF = 4
_QCHUNKS = 40
_QEDGES = _QCHUNKS * _CHUNK
_NSEC = E // _QEDGES
_SEC_PER_W = -(-_NSEC // _NW)
_N_PAD = 10240
_ROWS_PER_SUB = _N_PAD // _NS

_BR = 400
_NB = N // _BR
_BR2 = 2000
_NB2 = N // _BR2


def _sc_segment_sum(feat_a, edge_index):
    mesh = plsc.VectorSubcoreMesh(core_axis_name="c", subcore_axis_name="s")

    @functools.partial(
        pl.kernel,
        out_type=jax.ShapeDtypeStruct((_NC, _N_PAD, D), jnp.float32),
        mesh=mesh,
        scratch_types=[
            pltpu.VMEM_SHARED((_N_PAD, D), jnp.float32),
        ]
        + [pltpu.VMEM((_QEDGES,), jnp.int32) for _ in range(4)]
        + [pltpu.VMEM((_CHUNK, D), jnp.float32) for _ in range(_NBUF)]
        + [pltpu.VMEM((_CHUNK,), jnp.int32) for _ in range(2 * _NBUF)]
        + [pltpu.SemaphoreType.DMA for _ in range(2 * _NBUF + 2)],
    )
    def seg_sum(feat_hbm, edge_hbm, out_hbm, acc_sh, *bufs):
        srcq = bufs[0:2]
        dstq = bufs[2:4]
        rows = bufs[4:4 + _NBUF]
        srcb = bufs[4 + _NBUF:4 + 2 * _NBUF]
        dstb = bufs[4 + 2 * _NBUF:4 + 3 * _NBUF]
        sems = bufs[4 + 3 * _NBUF:]
        gsem = sems[:_NBUF]
        ssem = sems[_NBUF:2 * _NBUF]
        isem = sems[2 * _NBUF:]
        c = lax.axis_index("c")
        s = lax.axis_index("s")
        w = s * _NC + c

        def load_idx(u, qs, sync):
            off = u * _QEDGES
            for row, vm in ((0, srcq[qs]), (1, dstq[qs])):
                cp = pltpu.make_async_copy(
                    edge_hbm.at[row].at[pl.ds(off, _QEDGES)], vm, isem[qs])
                cp.start()
                if sync:
                    cp.wait()

        def wait_idx(u, qs):
            off = u * _QEDGES
            for row, vm in ((0, srcq[qs]), (1, dstq[qs])):
                pltpu.make_async_copy(
                    edge_hbm.at[row].at[pl.ds(off, _QEDGES)], vm,
                    isem[qs]).wait()

        load_idx(w, 0, False)

        @pl.loop(0, _CHUNK)
        def _(r):
            @pl.loop(0, D // 16)
            def _(k):
                rows[0][r, pl.ds(k * 16, 16)] = jnp.zeros((16,), jnp.float32)

        @pl.loop(0, _ROWS_PER_SUB // _CHUNK)
        def _(k):
            pltpu.sync_copy(
                rows[0], acc_sh.at[pl.ds(s * _ROWS_PER_SUB + k * _CHUNK, _CHUNK)])

        wait_idx(w, 0)

        def stage_idx(slab, buf, j):
            for k in range(_CHUNK // 16):
                buf[pl.ds(k * 16, 16)] = slab[pl.ds(j * _CHUNK + k * 16, 16)]

        def start_gather(sq, j, b):
            stage_idx(sq, srcb[b], j)
            pltpu.async_copy(feat_hbm.at[srcb[b]], rows[b], gsem[b])

        def wait_gather(b):
            pltpu.make_async_copy(feat_hbm.at[srcb[b]], rows[b],
                                  gsem[b]).wait()

        def start_scatter(dq, j, b):
            stage_idx(dq, dstb[b], j)
            pltpu.async_copy(rows[b], acc_sh.at[dstb[b]], ssem[b], add=True)

        def wait_scatter(b):
            pltpu.make_async_copy(rows[b], acc_sh.at[dstb[b]], ssem[b]).wait()

        for b in range(_NBUF - 1):
            start_gather(srcq[0], b, b)

        plsc.subcore_barrier()

        for t in range(_SEC_PER_W):
            qs = t % 2
            sq, dq = srcq[qs], dstq[qs]
            u = w + _NW * t

            @pl.when(u < _NSEC)
            def _():
                if t > 0:
                    wait_idx(u, qs)
                if t + 1 < _SEC_PER_W:
                    nxt = w + _NW * (t + 1)

                    @pl.when(nxt < _NSEC)
                    def _():
                        load_idx(nxt, 1 - qs, False)

                if t > 0:
                    for b in range(_NBUF - 1):
                        start_gather(sq, b, b)

                @pl.loop(0, _QCHUNKS // _NBUF)
                def _(g):
                    for b in range(_NBUF):
                        j = g * _NBUF + b
                        prev_slot = (b - 1) % _NBUF
                        wait_gather(b)

                        @pl.when(j >= 1)
                        def _():
                            wait_scatter(prev_slot)

                        @pl.when(j + _NBUF - 1 < _QCHUNKS)
                        def _():
                            start_gather(sq, j + _NBUF - 1, prev_slot)

                        start_scatter(dq, j, b)

                wait_scatter((_QCHUNKS - 1) % _NBUF)

        plsc.subcore_barrier()

        pltpu.sync_copy(
            acc_sh.at[pl.ds(s * _ROWS_PER_SUB, _ROWS_PER_SUB)],
            out_hbm.at[c].at[pl.ds(s * _ROWS_PER_SUB, _ROWS_PER_SUB)],
        )

    return seg_sum(feat_a, edge_index)


def _tc_partial(feat_a, deg, pm_pd, feat_b, W_prev, W_deg, W_fuse):
    def body(fa, dg, pp, fb, wp, wd, wf, out, fb_proj):
        i = pl.program_id(0)

        @pl.when(i == 0)
        def _():
            fb_proj[...] = jnp.dot(fb[...], wf[...],
                                   preferred_element_type=jnp.float32)

        acc = jnp.dot(fa[...], wp[...], preferred_element_type=jnp.float32)
        acc += jnp.dot(dg[...] * fa[...], wd[...],
                       preferred_element_type=jnp.float32)
        acc += jnp.dot(pp[...], fb_proj[...], preferred_element_type=jnp.float32)
        out[...] = acc

    return pl.pallas_call(
        body,
        grid=(_NB,),
        in_specs=[
            pl.BlockSpec((_BR, D), lambda i: (i, 0)),
            pl.BlockSpec((_BR, 1), lambda i: (i, 0)),
            pl.BlockSpec((_BR, M), lambda i: (i, 0)),
            pl.BlockSpec((M, D), lambda i: (0, 0)),
            pl.BlockSpec((D, D), lambda i: (0, 0)),
            pl.BlockSpec((D, D), lambda i: (0, 0)),
            pl.BlockSpec((D, D), lambda i: (0, 0)),
        ],
        out_specs=pl.BlockSpec((_BR, D), lambda i: (i, 0)),
        out_shape=jax.ShapeDtypeStruct((N, D), jnp.float32),
        scratch_shapes=[pltpu.VMEM((M, D), jnp.float32)],
    )(feat_a, deg, pm_pd, feat_b, W_prev, W_deg, W_fuse)


def _tc_finish(partial, agg2, W_gcn, bias, gamma, beta):
    def body(pt, a01, wg, b, g, bt, out, acc):
        i = pl.program_id(0)
        a = a01[0] + a01[1]
        r = pt[...] + jnp.dot(a, wg[...], preferred_element_type=jnp.float32)
        r = r + b[...]
        out[pl.ds(i * _BR2, _BR2), :] = r

        @pl.when(i == 0)
        def _():
            acc[...] = jnp.zeros_like(acc)

        acc[0:1, :] += jnp.sum(r, axis=0, keepdims=True)
        acc[1:2, :] += jnp.sum(r * r, axis=0, keepdims=True)

        @pl.when(i == _NB2 - 1)
        def _():
            mean = acc[0:1, :] * (1.0 / N)
            var = acc[1:2, :] * (1.0 / N) - mean * mean
            scale = g[...] * lax.rsqrt(var + 1e-5)
            shift = bt[...] - mean * scale
            out[...] = out[...] * scale + shift

    return pl.pallas_call(
        body,
        grid=(_NB2,),
        in_specs=[
            pl.BlockSpec((_BR2, D), lambda i: (i, 0)),
            pl.BlockSpec((_NC, _BR2, D), lambda i: (0, i, 0)),
            pl.BlockSpec((D, D), lambda i: (0, 0)),
            pl.BlockSpec((1, D), lambda i: (0, 0)),
            pl.BlockSpec((1, D), lambda i: (0, 0)),
            pl.BlockSpec((1, D), lambda i: (0, 0)),
        ],
        out_specs=pl.BlockSpec((N, D), lambda i: (0, 0)),
        out_shape=jax.ShapeDtypeStruct((N, D), jnp.float32),
        scratch_shapes=[pltpu.VMEM((8, D), jnp.float32)],
    )(partial, agg2, W_gcn, bias, gamma, beta)


def kernel(feat_a, feat_b, deg, pm_pd, edge_index, W_prev, b_prev, W_deg,
           b_deg, W_gcn, b_gcn, W_fuse, b_fuse, bn_gamma, bn_beta):
    agg2 = _sc_segment_sum(feat_a, edge_index)
    partial = _tc_partial(feat_a, deg, pm_pd, feat_b, W_prev, W_deg, W_fuse)
    bias = (b_prev + b_deg + b_gcn + b_fuse).reshape(1, D)
    return _tc_finish(partial, agg2, W_gcn, bias,
                      bn_gamma.reshape(1, D), bn_beta.reshape(1, D))
